# Initial kernel scaffold; baseline (speedup 1.0000x reference)
import jax
import jax.numpy as jnp
from jax.experimental import pallas as pl


def kernel(target_ids, context_ids, neg_ids, target_W, context_W):
    raise NotImplementedError("write your pallas kernel here")


# speedup vs baseline: 4.6924x; 4.6924x over previous
import functools

import jax
import jax.numpy as jnp
from jax import lax
from jax.experimental import pallas as pl
from jax.experimental.pallas import tpu as pltpu
from jax.experimental.pallas import tpu_sc as plsc

L = 16


def _sc_scores(B, K, D, C):
    info = plsc.get_sparse_core_info()
    NC, NS = info.num_cores, info.num_subcores
    NW = NC * NS
    assert B % (NW * C) == 0
    n_chunks = B // (NW * C)
    n_sub = D // L

    mesh = plsc.VectorSubcoreMesh(core_axis_name="c", subcore_axis_name="s")

    @functools.partial(
        pl.kernel,
        mesh=mesh,
        compiler_params=pltpu.CompilerParams(
            needs_layout_passes=False, use_tc_tiling_on_sc=False),
        out_type=[
            jax.ShapeDtypeStruct((B,), jnp.float32),
            jax.ShapeDtypeStruct((B * K,), jnp.float32),
        ],
        scratch_types=[
            pltpu.VMEM((C,), jnp.int32),
            pltpu.VMEM((C,), jnp.int32),
            pltpu.VMEM((C * K,), jnp.int32),
            pltpu.VMEM((C, D), jnp.float32),
            pltpu.VMEM((C, D), jnp.float32),
            pltpu.VMEM((C * K, D), jnp.float32),
            pltpu.VMEM((C,), jnp.float32),
            pltpu.VMEM((C * K,), jnp.float32),
            pltpu.SemaphoreType.DMA,
            pltpu.SemaphoreType.DMA,
            pltpu.SemaphoreType.DMA,
        ],
    )
    def sc_kernel(tids_hbm, cids_hbm, nids_hbm, tW_hbm, cW_hbm,
                  pos_hbm, neg_hbm,
                  tid_v, cid_v, nid_v, trows, crows, nrows,
                  posbuf, negbuf, sem_t, sem_c, sem_n):
        wid = lax.axis_index("s") * NC + lax.axis_index("c")
        base = wid * (n_chunks * C)

        def chunk_body(ch, _):
            c0 = base + ch * C
            pltpu.sync_copy(tids_hbm.at[pl.ds(c0, C)], tid_v)
            pltpu.sync_copy(cids_hbm.at[pl.ds(c0, C)], cid_v)
            pltpu.sync_copy(nids_hbm.at[pl.ds(c0 * K, C * K)], nid_v)
            ct = pltpu.async_copy(tW_hbm.at[tid_v], trows, sem_t)
            cc = pltpu.async_copy(cW_hbm.at[cid_v], crows, sem_c)
            cn = pltpu.async_copy(cW_hbm.at[nid_v], nrows, sem_n)
            ct.wait()
            cc.wait()
            cn.wait()

            lane = lax.iota(jnp.int32, L)
            first = lane == 0

            dnums = lax.GatherDimensionNumbers(
                offset_dims=(), collapsed_slice_dims=(0,),
                start_index_map=(0,))

            def shuffle(v, idx):
                return lax.gather(
                    v, idx[:, None], dimension_numbers=dnums,
                    slice_sizes=(1,),
                    mode=lax.GatherScatterMode.PROMISE_IN_BOUNDS)

            def hsum(v):
                for s in (8, 4, 2, 1):
                    v = v + shuffle(v, lane ^ s)
                return v

            def row_body(i, _):
                t = [trows[i, pl.ds(j * L, L)] for j in range(n_sub)]
                acc = t[0] * crows[i, pl.ds(0, L)]
                for j in range(1, n_sub):
                    acc = acc + t[j] * crows[i, pl.ds(j * L, L)]
                plsc.store_scatter(posbuf, [lane * 0 + i],
                                   hsum(acc), mask=first)
                for k in range(K):
                    r = i * K + k
                    acc = t[0] * nrows[r, pl.ds(0, L)]
                    for j in range(1, n_sub):
                        acc = acc + t[j] * nrows[r, pl.ds(j * L, L)]
                    plsc.store_scatter(negbuf, [lane * 0 + r],
                                       hsum(acc), mask=first)
                return 0

            lax.fori_loop(0, C, row_body, 0)
            pltpu.sync_copy(posbuf, pos_hbm.at[pl.ds(c0, C)])
            pltpu.sync_copy(negbuf, neg_hbm.at[pl.ds(c0 * K, C * K)])
            return 0

        lax.fori_loop(0, n_chunks, chunk_body, 0)

    return sc_kernel


def _tc_loss_kernel(pos_ref, neg_ref, out_ref):
    p = pos_ref[...]
    n = neg_ref[...]
    pos_ls = jnp.minimum(p, 0.0) - jnp.log1p(jnp.exp(-jnp.abs(p)))
    m = -n
    neg_ls = jnp.minimum(m, 0.0) - jnp.log1p(jnp.exp(-jnp.abs(m)))
    total = jnp.sum(pos_ls) + jnp.sum(neg_ls)
    out_ref[0, 0] = -total / p.size


def kernel(target_ids, context_ids, neg_ids, target_W, context_W):
    B, K = neg_ids.shape
    V, D = target_W.shape
    neg_flat = neg_ids.reshape(B * K)

    sc = _sc_scores(B, K, D, C=64)
    pos_score, neg_score = sc(target_ids, context_ids, neg_flat,
                              target_W, context_W)

    loss = pl.pallas_call(
        _tc_loss_kernel,
        out_shape=jax.ShapeDtypeStruct((1, 1), jnp.float32),
        out_specs=pl.BlockSpec(memory_space=pltpu.SMEM),
    )(pos_score.reshape(B // 128, 128), neg_score.reshape(B * K // 128, 128))
    return loss[0, 0]
